# baseline (speedup 1.0000x reference)
import functools

import jax
import jax.numpy as jnp
from jax import lax
from jax.experimental import pallas as pl
from jax.experimental.pallas import tpu as pltpu
from jax.experimental.pallas import tpu_sc as plsc

_NV = 8192
_D = 256
_NTOK = 8192
_BETA = 0.25
_M_BLK = 128

_NW = 32
_B_PER_W = _NTOK // _NW
_CH = 128


def _argmin_body(xi_ref, x_ref, kwt_ref, ej_ref, idx_ref):
    s = jnp.dot(x_ref[...], kwt_ref[...], preferred_element_type=jnp.float32)
    d = (xi_ref[...] + ej_ref[0:1, :]) - 2.0 * s
    bmin = jnp.min(d, axis=1, keepdims=True)
    ids = lax.broadcasted_iota(jnp.int32, d.shape, 1)
    sel = jnp.where(d == bmin, ids, _NV)
    idx_ref[...] = jnp.min(sel, axis=1, keepdims=True)


def _argmin_call(xi, x_flat, kwt, ej2):
    return pl.pallas_call(
        _argmin_body,
        grid=(_NTOK // _M_BLK,),
        in_specs=[
            pl.BlockSpec((_M_BLK, 1), lambda m: (m, 0)),
            pl.BlockSpec((_M_BLK, _D), lambda m: (m, 0)),
            pl.BlockSpec((_D, _NV), lambda m: (0, 0)),
            pl.BlockSpec((8, _NV), lambda m: (0, 0)),
        ],
        out_specs=pl.BlockSpec((_M_BLK, 1), lambda m: (m, 0)),
        out_shape=jax.ShapeDtypeStruct((_NTOK, 1), jnp.int32),
    )(xi, x_flat, kwt, ej2)


def _sc_gather(table, idx):
    mesh = plsc.VectorSubcoreMesh(core_axis_name="c", subcore_axis_name="s")

    @functools.partial(
        pl.kernel,
        mesh=mesh,
        out_type=jax.ShapeDtypeStruct((_NTOK, _D), jnp.float32),
        scratch_types=[
            pltpu.VMEM((_CH,), jnp.int32),
            pltpu.VMEM((_CH, _D), jnp.float32),
            pltpu.SemaphoreType.DMA,
        ],
    )
    def _g(table_hbm, idx_hbm, out_hbm, idx_v, rows_v, sem):
        wid = lax.axis_index("s") * 2 + lax.axis_index("c")
        base = wid * _B_PER_W
        for ci in range(_B_PER_W // _CH):
            off = base + ci * _CH
            pltpu.sync_copy(idx_hbm.at[pl.ds(off, _CH)], idx_v)
            pltpu.async_copy(table_hbm.at[idx_v], rows_v, sem).wait()
            pltpu.sync_copy(rows_v, out_hbm.at[pl.ds(off, _CH)])

    return _g(table, idx)


_E_BLK = 1024


def _epilogue_body(xqp_ref, x_ref, v_ref, st_ref, grad_ref, loss_ref):
    i = pl.program_id(0)
    xq = jnp.dot(xqp_ref[...], v_ref[...], preferred_element_type=jnp.float32)
    xb = x_ref[...]
    grad_ref[...] = xq
    st_ref[...] = xb + (xq - xb)
    part = jnp.sum((xq - xb) ** 2)

    @pl.when(i == 0)
    def _init():
        loss_ref[...] = jnp.zeros((1, 1), jnp.float32)

    loss_ref[...] = loss_ref[...] + part

    @pl.when(i == pl.num_programs(0) - 1)
    def _fin():
        m = loss_ref[...] / (_NTOK * _D)
        loss_ref[...] = m + _BETA * m


def _epilogue_call(xq_pre, x_flat, value_weight):
    return pl.pallas_call(
        _epilogue_body,
        grid=(_NTOK // _E_BLK,),
        in_specs=[
            pl.BlockSpec((_E_BLK, _D), lambda i: (i, 0)),
            pl.BlockSpec((_E_BLK, _D), lambda i: (i, 0)),
            pl.BlockSpec((_D, _D), lambda i: (0, 0)),
        ],
        out_specs=[
            pl.BlockSpec((_E_BLK, _D), lambda i: (i, 0)),
            pl.BlockSpec((_E_BLK, _D), lambda i: (i, 0)),
            pl.BlockSpec((1, 1), lambda i: (0, 0)),
        ],
        out_shape=[
            jax.ShapeDtypeStruct((_NTOK, _D), jnp.float32),
            jax.ShapeDtypeStruct((_NTOK, _D), jnp.float32),
            jax.ShapeDtypeStruct((1, 1), jnp.float32),
        ],
    )(xq_pre, x_flat, value_weight)


def kernel(x, key_weight, value_weight):
    x_t = jnp.transpose(x, (0, 2, 3, 1))
    b, h, w, c = x_t.shape
    x_flat = x_t.reshape(-1, c)
    xi = jnp.sum(x_flat ** 2, axis=1, keepdims=True)
    ej = jnp.sum(key_weight ** 2, axis=1)
    ej2 = jnp.broadcast_to(ej[None, :], (8, _NV))
    idx2 = _argmin_call(xi, x_flat, key_weight.T, ej2)
    idx = idx2.reshape(_NTOK)
    xq_pre = _sc_gather(key_weight, idx)
    st_flat, grad_flat, loss11 = _epilogue_call(xq_pre, x_flat, value_weight)
    y_st = jnp.transpose(st_flat.reshape(b, h, w, c), (0, 3, 1, 2))
    y_grad = jnp.transpose(grad_flat.reshape(b, h, w, c), (0, 3, 1, 2))
    return (y_st, y_grad, idx, loss11[0, 0])
